# Initial kernel scaffold; baseline (speedup 1.0000x reference)
import functools

import jax
import jax.numpy as jnp
from jax import lax
from jax.experimental import pallas as pl
from jax.experimental.pallas import tpu as pltpu
from jax.experimental.pallas import tpu_sc as plsc

_VOCAB = 100000
_S = 200
_D = 64
_B = 4096

_NC = 2
_NS = 16
_NW = _NC * _NS

_SEQ_PER_W = _B // _NW
_NCHUNK = _SEQ_PER_W
_HALF = _S // 2
_NBUF = 4

_mesh = plsc.VectorSubcoreMesh(core_axis_name="c", subcore_axis_name="s")


@functools.partial(
    pl.kernel,
    mesh=_mesh,
    out_type=jax.ShapeDtypeStruct((_NW * _NCHUNK, 2, _HALF, _D), jnp.float32),
    scratch_types=[
        pltpu.VMEM((2, _HALF, _D), jnp.float32),
        pltpu.VMEM((_NCHUNK, 2, _HALF), jnp.int32),
    ]
    + [pltpu.VMEM((2, _HALF, _D), jnp.float32) for _ in range(_NBUF)]
    + [pltpu.SemaphoreType.DMA for _ in range(2 * _NBUF)],
)
def _embed_pe(x_hbm, emb_hbm, pos_hbm, out_hbm, pos_v, idx_v,
              b0, b1, b2, b3, g0, g1, g2, g3, s0, s1, s2, s3):
    bufs = [b0, b1, b2, b3]
    gsems = [g0, g1, g2, g3]
    ssems = [s0, s1, s2, s3]

    cid = lax.axis_index("c")
    sid = lax.axis_index("s")
    wid = sid * _NC + cid

    pltpu.sync_copy(pos_hbm, pos_v)
    pltpu.sync_copy(x_hbm.at[wid], idx_v)

    def issue_gather(c, b):
        pltpu.async_copy(emb_hbm.at[idx_v.at[c, 0]], bufs[b].at[0], gsems[b])
        pltpu.async_copy(emb_hbm.at[idx_v.at[c, 1]], bufs[b].at[1], gsems[b])

    def wait_gather(b):
        pltpu.make_async_copy(pos_hbm, bufs[b], gsems[b]).wait()

    def issue_scatter(c, b):
        pltpu.async_copy(bufs[b], out_hbm.at[wid * _NCHUNK + c], ssems[b])

    def wait_scatter(b):
        pltpu.make_async_copy(bufs[b], out_hbm.at[0], ssems[b]).wait()

    def add_pos(b):
        buf = bufs[b]

        def row(r, carry):
            for k in range(2):
                for j in range(0, _D, 16):
                    v = pos_v[k, r, pl.ds(j, 16)]
                    plsc.addupdate(buf.at[k, r, pl.ds(j, 16)], v)
            return carry

        lax.fori_loop(0, _HALF, row, 0)

    for b in range(_NBUF - 1):
        issue_gather(b, b)

    def outer(i, carry):
        for b in range(_NBUF):
            c = i * _NBUF + b
            bp = (b + _NBUF - 1) % _NBUF

            @pl.when(c >= 1)
            def _():
                wait_scatter(bp)

            @pl.when(c + (_NBUF - 1) < _NCHUNK)
            def _():
                issue_gather(c + (_NBUF - 1), bp)

            wait_gather(b)
            add_pos(b)
            issue_scatter(c, b)
        return carry

    lax.fori_loop(0, _NCHUNK // _NBUF, outer, 0)

    wait_scatter(_NBUF - 1)


def kernel(x, emb_table, pos_table):
    x4 = x.astype(jnp.int32).reshape(_NW, _NCHUNK, 2, _HALF)
    pos3 = pos_table.reshape(2, _HALF, _D)
    out = _embed_pe(x4, emb_table, pos3)
    return out.reshape(_B, _S, _D)


# speedup vs baseline: 3.4774x; 3.4774x over previous
import functools

import jax
import jax.numpy as jnp
from jax import lax
from jax.experimental import pallas as pl
from jax.experimental.pallas import tpu as pltpu
from jax.experimental.pallas import tpu_sc as plsc

_VOCAB = 100000
_S = 200
_D = 64
_B = 4096

_NC = 2
_NS = 16
_NW = _NC * _NS

_SEQ_PER_W = _B // _NW
_NCHUNK = _SEQ_PER_W
_HALF = _S // 2
_NBUF = 4

_mesh = plsc.VectorSubcoreMesh(core_axis_name="c", subcore_axis_name="s")


@functools.partial(
    pl.kernel,
    mesh=_mesh,
    out_type=jax.ShapeDtypeStruct((_NW * _NCHUNK, 2, _HALF, _D), jnp.float32),
    scratch_types=[
        pltpu.VMEM((2, _HALF, _D), jnp.float32),
        pltpu.VMEM((_NCHUNK, 2, _HALF), jnp.int32),
    ]
    + [pltpu.VMEM((2, _HALF, _D), jnp.float32) for _ in range(_NBUF)]
    + [pltpu.SemaphoreType.DMA for _ in range(2 * _NBUF)],
    compiler_params=pltpu.CompilerParams(use_tc_tiling_on_sc=False),
)
def _embed_pe(x_hbm, emb_hbm, pos_hbm, out_hbm, pos_v, idx_v,
              b0, b1, b2, b3, g0, g1, g2, g3, s0, s1, s2, s3):
    bufs = [b0, b1, b2, b3]
    gsems = [g0, g1, g2, g3]
    ssems = [s0, s1, s2, s3]

    cid = lax.axis_index("c")
    sid = lax.axis_index("s")
    wid = sid * _NC + cid

    pltpu.sync_copy(pos_hbm, pos_v)
    pltpu.sync_copy(x_hbm.at[wid], idx_v)

    def issue_gather(c, b):
        pltpu.async_copy(emb_hbm.at[idx_v.at[c, 0]], bufs[b].at[0], gsems[b])
        pltpu.async_copy(emb_hbm.at[idx_v.at[c, 1]], bufs[b].at[1], gsems[b])

    def wait_gather(b):
        pltpu.make_async_copy(pos_hbm, bufs[b], gsems[b]).wait()

    def issue_scatter(c, b):
        pltpu.async_copy(bufs[b], out_hbm.at[wid * _NCHUNK + c], ssems[b])

    def wait_scatter(b):
        pltpu.make_async_copy(bufs[b], out_hbm.at[0], ssems[b]).wait()

    def add_pos(b):
        buf = bufs[b]

        def row(r, carry):
            for k in range(2):
                for j in range(0, _D, 16):
                    v = pos_v[k, r, pl.ds(j, 16)]
                    plsc.addupdate(buf.at[k, r, pl.ds(j, 16)], v)
            return carry

        lax.fori_loop(0, _HALF, row, 0)

    for b in range(_NBUF - 1):
        issue_gather(b, b)

    def outer(i, carry):
        for b in range(_NBUF):
            c = i * _NBUF + b
            bp = (b + _NBUF - 1) % _NBUF

            @pl.when(c >= 1)
            def _():
                wait_scatter(bp)

            @pl.when(c + (_NBUF - 1) < _NCHUNK)
            def _():
                issue_gather(c + (_NBUF - 1), bp)

            wait_gather(b)
            add_pos(b)
            issue_scatter(c, b)
        return carry

    lax.fori_loop(0, _NCHUNK // _NBUF, outer, 0)

    wait_scatter(_NBUF - 1)


def kernel(x, emb_table, pos_table):
    x4 = x.astype(jnp.int32).reshape(_NW, _NCHUNK, 2, _HALF)
    pos3 = pos_table.reshape(2, _HALF, _D)
    out = _embed_pe(x4, emb_table, pos3)
    return out.reshape(_B, _S, _D)


# speedup vs baseline: 3.9815x; 1.1450x over previous
import functools

import jax
import jax.numpy as jnp
from jax import lax
from jax.experimental import pallas as pl
from jax.experimental.pallas import tpu as pltpu
from jax.experimental.pallas import tpu_sc as plsc

_VOCAB = 100000
_S = 200
_D = 64
_B = 4096

_NC = 2
_NS = 16
_NW = _NC * _NS

_SEQ_PER_W = _B // _NW
_NCHUNK = _SEQ_PER_W
_NBUF = 4

_mesh = plsc.VectorSubcoreMesh(core_axis_name="c", subcore_axis_name="s")


@functools.partial(
    pl.kernel,
    mesh=_mesh,
    out_type=jax.ShapeDtypeStruct((_B, _S, _D), jnp.float32),
    scratch_types=[
        pltpu.VMEM((_S, _D), jnp.float32),
        pltpu.VMEM((_NCHUNK, _S), jnp.int32),
    ]
    + [pltpu.VMEM((_S, _D), jnp.float32) for _ in range(_NBUF)]
    + [pltpu.SemaphoreType.DMA for _ in range(2 * _NBUF)],
    compiler_params=pltpu.CompilerParams(use_tc_tiling_on_sc=False),
)
def _embed_pe(x_hbm, emb_hbm, pos_hbm, out_hbm, pos_v, idx_v,
              b0, b1, b2, b3, g0, g1, g2, g3, s0, s1, s2, s3):
    bufs = [b0, b1, b2, b3]
    gsems = [g0, g1, g2, g3]
    ssems = [s0, s1, s2, s3]

    cid = lax.axis_index("c")
    sid = lax.axis_index("s")
    wid = sid * _NC + cid

    pltpu.sync_copy(pos_hbm, pos_v)
    pltpu.sync_copy(x_hbm.at[pl.ds(wid * _NCHUNK, _NCHUNK)], idx_v)

    def issue_gather(c, b):
        pltpu.async_copy(emb_hbm.at[idx_v.at[c]], bufs[b], gsems[b])

    def wait_gather(b):
        pltpu.make_async_copy(pos_hbm, bufs[b], gsems[b]).wait()

    def issue_scatter(c, b):
        pltpu.async_copy(bufs[b], out_hbm.at[wid * _NCHUNK + c], ssems[b])

    def wait_scatter(b):
        pltpu.make_async_copy(bufs[b], out_hbm.at[0], ssems[b]).wait()

    def add_pos(b):
        buf = bufs[b]

        def row(r, carry):
            for rr in range(2):
                for j in range(0, _D, 16):
                    v = pos_v[2 * r + rr, pl.ds(j, 16)]
                    plsc.addupdate(buf.at[2 * r + rr, pl.ds(j, 16)], v)
            return carry

        lax.fori_loop(0, _S // 2, row, 0)

    for b in range(_NBUF - 1):
        issue_gather(b, b)

    def outer(i, carry):
        for b in range(_NBUF):
            c = i * _NBUF + b
            bp = (b + _NBUF - 1) % _NBUF

            @pl.when(c >= 1)
            def _():
                wait_scatter(bp)

            @pl.when(c + (_NBUF - 1) < _NCHUNK)
            def _():
                issue_gather(c + (_NBUF - 1), bp)

            wait_gather(b)
            add_pos(b)
            issue_scatter(c, b)
        return carry

    lax.fori_loop(0, _NCHUNK // _NBUF, outer, 0)

    wait_scatter(_NBUF - 1)


def kernel(x, emb_table, pos_table):
    return _embed_pe(x.astype(jnp.int32), emb_table, pos_table)


# speedup vs baseline: 4.4494x; 1.1175x over previous
import functools

import jax
import jax.numpy as jnp
from jax import lax
from jax.experimental import pallas as pl
from jax.experimental.pallas import tpu as pltpu
from jax.experimental.pallas import tpu_sc as plsc

_VOCAB = 100000
_S = 200
_D = 64
_B = 4096

_NC = 2
_NS = 16
_NW = _NC * _NS

_NSEQ = _B // _NW
_SA = 104
_SB = _S - _SA

_mesh = plsc.VectorSubcoreMesh(core_axis_name="c", subcore_axis_name="s")


@functools.partial(
    pl.kernel,
    mesh=_mesh,
    out_type=jax.ShapeDtypeStruct((_B, _S, _D), jnp.float32),
    scratch_types=[
        pltpu.VMEM((_SA, _D), jnp.float32),
        pltpu.VMEM((_SB, _D), jnp.float32),
        pltpu.VMEM((_NSEQ, _SA), jnp.int32),
        pltpu.VMEM((_NSEQ, _SB), jnp.int32),
        pltpu.VMEM((_SA, 2 * _D), jnp.float32),
        pltpu.VMEM((_SB, 2 * _D), jnp.float32),
        pltpu.VMEM((_SA, _D), jnp.float32),
        pltpu.VMEM((_SB, _D), jnp.float32),
        pltpu.SemaphoreType.DMA,
        pltpu.SemaphoreType.DMA,
        pltpu.SemaphoreType.DMA,
        pltpu.SemaphoreType.DMA,
    ],
    compiler_params=pltpu.CompilerParams(use_tc_tiling_on_sc=True),
)
def _embed_pe(xa_hbm, xb_hbm, emb2_hbm, posa_hbm, posb_hbm, out_hbm,
              pos_va, pos_vb, ia_v, ib_v, gba, gbb, sba, sbb,
              ga_sem, gb_sem, sa_sem, sb_sem):
    cid = lax.axis_index("c")
    sid = lax.axis_index("s")
    wid = sid * _NC + cid
    seq0 = wid * _NSEQ

    pltpu.sync_copy(posa_hbm, pos_va)
    pltpu.sync_copy(posb_hbm, pos_vb)
    pltpu.sync_copy(xa_hbm.at[pl.ds(seq0, _NSEQ)], ia_v)
    pltpu.sync_copy(xb_hbm.at[pl.ds(seq0, _NSEQ)], ib_v)

    def issue_gather_a(s):
        pltpu.async_copy(emb2_hbm.at[ia_v.at[s]], gba, ga_sem)

    def issue_gather_b(s):
        pltpu.async_copy(emb2_hbm.at[ib_v.at[s]], gbb, gb_sem)

    def wait_gather_a():
        pltpu.make_async_copy(emb2_hbm.at[pl.ds(0, _SA)], gba, ga_sem).wait()

    def wait_gather_b():
        pltpu.make_async_copy(emb2_hbm.at[pl.ds(0, _SB)], gbb, gb_sem).wait()

    def issue_scatter_a(s):
        pltpu.async_copy(sba, out_hbm.at[seq0 + s, pl.ds(0, _SA)], sa_sem)

    def issue_scatter_b(s):
        pltpu.async_copy(sbb, out_hbm.at[seq0 + s, pl.ds(_SA, _SB)], sb_sem)

    def wait_scatter_a():
        pltpu.make_async_copy(sba, out_hbm.at[0, pl.ds(0, _SA)], sa_sem).wait()

    def wait_scatter_b():
        pltpu.make_async_copy(sbb, out_hbm.at[0, pl.ds(_SA, _SB)], sb_sem).wait()

    def add_rows(gb, sb, pos_v, nrows):
        def row(r, carry):
            for j in range(0, _D, 16):
                v = gb[r, pl.ds(j, 16)] + pos_v[r, pl.ds(j, 16)]
                sb[r, pl.ds(j, 16)] = v
            return carry

        lax.fori_loop(0, nrows, row, 0)

    issue_gather_a(0)

    def seq_body(s, carry):
        issue_gather_b(s)
        wait_gather_a()

        @pl.when(s >= 1)
        def _():
            wait_scatter_a()
        add_rows(gba, sba, pos_va, _SA)
        issue_scatter_a(s)

        @pl.when(s + 1 < _NSEQ)
        def _():
            issue_gather_a(s + 1)
        wait_gather_b()

        @pl.when(s >= 1)
        def _():
            wait_scatter_b()
        add_rows(gbb, sbb, pos_vb, _SB)
        issue_scatter_b(s)
        return carry

    lax.fori_loop(0, _NSEQ, seq_body, 0)
    wait_scatter_a()
    wait_scatter_b()


def kernel(x, emb_table, pos_table):
    xi = x.astype(jnp.int32)
    emb2 = jnp.concatenate([emb_table, emb_table], axis=1)
    return _embed_pe(xi[:, :_SA], xi[:, _SA:], emb2,
                     pos_table[:_SA], pos_table[_SA:])
